# baseline (speedup 1.0000x reference)
import functools
import math

import jax
import jax.numpy as jnp
from jax import lax
from jax.experimental import pallas as pl
from jax.experimental.pallas import tpu as pltpu
from jax.experimental.pallas import tpu_sc as plsc

_F32 = jnp.float32


def _ssilu(v):
    return jax.nn.silu(v) * (1.0 / 0.6)


def _pick_block(n, prefs):
    for p in prefs:
        if n % p == 0:
            return p
    return n


def _proj_body(x_ref, w1_ref, w2_ref, a_ref, b_ref):
    xb = x_ref[...]
    a_ref[...] = jnp.dot(xb, w1_ref[...], preferred_element_type=_F32)
    b_ref[...] = jnp.dot(xb, w2_ref[...], preferred_element_type=_F32)


def _node_proj(x, w1, w2):
    n, d = x.shape
    nb = _pick_block(n, (400, 200, 100, 8))
    grid = (n // nb,)
    return pl.pallas_call(
        _proj_body,
        grid=grid,
        in_specs=[
            pl.BlockSpec((nb, d), lambda i: (i, 0)),
            pl.BlockSpec((d, d), lambda i: (0, 0)),
            pl.BlockSpec((d, d), lambda i: (0, 0)),
        ],
        out_specs=[
            pl.BlockSpec((nb, d), lambda i: (i, 0)),
            pl.BlockSpec((nb, d), lambda i: (i, 0)),
        ],
        out_shape=[
            jax.ShapeDtypeStruct((n, d), _F32),
            jax.ShapeDtypeStruct((n, d), _F32),
        ],
    )(x, w1, w2)


def _msg_body(ga_ref, gb_ref, ea_ref, wr_ref, w3_ref, wm_ref, out_ref):
    rbf = _ssilu(jnp.dot(ea_ref[...], wr_ref[...], preferred_element_type=_F32))
    c = jnp.dot(rbf, w3_ref[...], preferred_element_type=_F32)
    m = _ssilu(ga_ref[...] + gb_ref[...] + c)
    out_ref[...] = _ssilu(jnp.dot(m, wm_ref[...], preferred_element_type=_F32))


def _edge_msg(ga, gb, ea, w_rbf, w3, w_msg):
    e, d = ga.shape
    de = ea.shape[1]
    eb = _pick_block(e, (512, 256, 128, 16))
    grid = (e // eb,)
    return pl.pallas_call(
        _msg_body,
        grid=grid,
        in_specs=[
            pl.BlockSpec((eb, d), lambda i: (i, 0)),
            pl.BlockSpec((eb, d), lambda i: (i, 0)),
            pl.BlockSpec((eb, de), lambda i: (i, 0)),
            pl.BlockSpec((de, de), lambda i: (0, 0)),
            pl.BlockSpec((de, d), lambda i: (0, 0)),
            pl.BlockSpec((d, d), lambda i: (0, 0)),
        ],
        out_specs=pl.BlockSpec((eb, d), lambda i: (i, 0)),
        out_shape=jax.ShapeDtypeStruct((e, d), _F32),
    )(ga, gb, ea, w_rbf, w3, w_msg)


def _atom_body(scale, agg_ref, x_ref, w1_ref, w2_ref, wres_ref, out_ref):
    h = _ssilu(jnp.dot(agg_ref[...] * scale, w1_ref[...], preferred_element_type=_F32))
    h = _ssilu(jnp.dot(h, w2_ref[...], preferred_element_type=_F32))
    out_ref[...] = (x_ref[...] + _ssilu(
        jnp.dot(h, wres_ref[...], preferred_element_type=_F32)
    )) * (1.0 / math.sqrt(2.0))


def _atom_update(agg, x, w1, w2, wres, scale):
    n, d = x.shape
    nb = _pick_block(n, (400, 200, 100, 8))
    grid = (n // nb,)
    return pl.pallas_call(
        functools.partial(_atom_body, scale),
        grid=grid,
        in_specs=[
            pl.BlockSpec((nb, d), lambda i: (i, 0)),
            pl.BlockSpec((nb, d), lambda i: (i, 0)),
            pl.BlockSpec((d, d), lambda i: (0, 0)),
            pl.BlockSpec((d, d), lambda i: (0, 0)),
            pl.BlockSpec((d, d), lambda i: (0, 0)),
        ],
        out_specs=pl.BlockSpec((nb, d), lambda i: (i, 0)),
        out_shape=jax.ShapeDtypeStruct((n, d), _F32),
    )(agg, x, w1, w2, wres)


def _gather_rows(a, b, src, dst):
    return jnp.take(a, src, axis=0), jnp.take(b, dst, axis=0)


def _scatter_add(msg, dst, n):
    return jnp.zeros((n, msg.shape[1]), dtype=msg.dtype).at[dst].add(msg)


def kernel(x, edge_index, edge_attr, W_rbf, W_edge, W_msg, W_atom1, W_atom2, W_res):
    n, d = x.shape
    e = edge_index.shape[1]
    src = edge_index[0].astype(jnp.int32)
    dst = edge_index[1].astype(jnp.int32)
    w1 = W_edge[:d]
    w2 = W_edge[d:2 * d]
    w3 = W_edge[2 * d:]
    a, b = _node_proj(x, w1, w2)
    ga, gb = _gather_rows(a, b, src, dst)
    msg = _edge_msg(ga, gb, edge_attr, W_rbf, w3, W_msg)
    agg = _scatter_add(msg, dst, n)
    scale = 1.0 / math.sqrt(e / n)
    return _atom_update(agg, x, W_atom1, W_atom2, W_res, scale)


# speedup vs baseline: 1.7941x; 1.7941x over previous
import functools
import math

import jax
import jax.numpy as jnp
from jax import lax
from jax.experimental import pallas as pl
from jax.experimental.pallas import tpu as pltpu
from jax.experimental.pallas import tpu_sc as plsc

_F32 = jnp.float32


def _ssilu(v):
    return jax.nn.silu(v) * (1.0 / 0.6)


def _pick_block(n, prefs):
    for p in prefs:
        if n % p == 0:
            return p
    return n


def _proj_body(x_ref, w1_ref, w2_ref, a_ref, b_ref):
    xb = x_ref[...]
    a_ref[...] = jnp.dot(xb, w1_ref[...], preferred_element_type=_F32)
    b_ref[...] = jnp.dot(xb, w2_ref[...], preferred_element_type=_F32)


def _node_proj(x, w1, w2):
    n, d = x.shape
    nb = _pick_block(n, (400, 200, 100, 8))
    grid = (n // nb,)
    return pl.pallas_call(
        _proj_body,
        grid=grid,
        in_specs=[
            pl.BlockSpec((nb, d), lambda i: (i, 0)),
            pl.BlockSpec((d, d), lambda i: (0, 0)),
            pl.BlockSpec((d, d), lambda i: (0, 0)),
        ],
        out_specs=[
            pl.BlockSpec((nb, d), lambda i: (i, 0)),
            pl.BlockSpec((nb, d), lambda i: (i, 0)),
        ],
        out_shape=[
            jax.ShapeDtypeStruct((n, d), _F32),
            jax.ShapeDtypeStruct((n, d), _F32),
        ],
    )(x, w1, w2)


def _msg_body(ga_ref, gb_ref, ea_ref, wr_ref, w3_ref, wm_ref, out_ref):
    rbf = _ssilu(jnp.dot(ea_ref[...], wr_ref[...], preferred_element_type=_F32))
    c = jnp.dot(rbf, w3_ref[...], preferred_element_type=_F32)
    m = _ssilu(ga_ref[...] + gb_ref[...] + c)
    out_ref[...] = _ssilu(jnp.dot(m, wm_ref[...], preferred_element_type=_F32))


def _edge_msg(ga, gb, ea, w_rbf, w3, w_msg):
    e, d = ga.shape
    de = ea.shape[1]
    eb = _pick_block(e, (512, 256, 128, 16))
    grid = (e // eb,)
    return pl.pallas_call(
        _msg_body,
        grid=grid,
        in_specs=[
            pl.BlockSpec((eb, d), lambda i: (i, 0)),
            pl.BlockSpec((eb, d), lambda i: (i, 0)),
            pl.BlockSpec((eb, de), lambda i: (i, 0)),
            pl.BlockSpec((de, de), lambda i: (0, 0)),
            pl.BlockSpec((de, d), lambda i: (0, 0)),
            pl.BlockSpec((d, d), lambda i: (0, 0)),
        ],
        out_specs=pl.BlockSpec((eb, d), lambda i: (i, 0)),
        out_shape=jax.ShapeDtypeStruct((e, d), _F32),
    )(ga, gb, ea, w_rbf, w3, w_msg)


def _atom_body(scale, agg_ref, x_ref, w1_ref, w2_ref, wres_ref, out_ref):
    h = _ssilu(jnp.dot(agg_ref[...] * scale, w1_ref[...], preferred_element_type=_F32))
    h = _ssilu(jnp.dot(h, w2_ref[...], preferred_element_type=_F32))
    out_ref[...] = (x_ref[...] + _ssilu(
        jnp.dot(h, wres_ref[...], preferred_element_type=_F32)
    )) * (1.0 / math.sqrt(2.0))


def _atom_update(agg, x, w1, w2, wres, scale):
    n, d = x.shape
    nb = _pick_block(n, (400, 200, 100, 8))
    grid = (n // nb,)
    return pl.pallas_call(
        functools.partial(_atom_body, scale),
        grid=grid,
        in_specs=[
            pl.BlockSpec((nb, d), lambda i: (i, 0)),
            pl.BlockSpec((nb, d), lambda i: (i, 0)),
            pl.BlockSpec((d, d), lambda i: (0, 0)),
            pl.BlockSpec((d, d), lambda i: (0, 0)),
            pl.BlockSpec((d, d), lambda i: (0, 0)),
        ],
        out_specs=pl.BlockSpec((nb, d), lambda i: (i, 0)),
        out_shape=jax.ShapeDtypeStruct((n, d), _F32),
    )(agg, x, w1, w2, wres)


def _sc_gather(a, b, src, dst):
    e = src.shape[0]
    d = a.shape[1]
    nw = 32
    epw = e // nw
    c = 80
    nch = epw // c
    assert e % nw == 0 and epw % c == 0 and nch % 2 == 1
    mesh = plsc.VectorSubcoreMesh(core_axis_name="c", subcore_axis_name="s")

    @functools.partial(
        pl.kernel,
        mesh=mesh,
        out_type=[
            jax.ShapeDtypeStruct((e, d), _F32),
            jax.ShapeDtypeStruct((e, d), _F32),
        ],
        scratch_types=[
            pltpu.VMEM((2, c), jnp.int32),
            pltpu.VMEM((2, c), jnp.int32),
            pltpu.VMEM((2, c, d), _F32),
            pltpu.VMEM((2, c, d), _F32),
            pltpu.SemaphoreType.DMA,
            pltpu.SemaphoreType.DMA,
            pltpu.SemaphoreType.DMA,
            pltpu.SemaphoreType.DMA,
        ],
    )
    def k(a_hbm, b_hbm, src_hbm, dst_hbm, ga_hbm, gb_hbm,
          idxs, idxd, bufa, bufb, sa0, sa1, sb0, sb1):
        wid = lax.axis_index("s") * 2 + lax.axis_index("c")
        base = wid * epw
        sa = (sa0, sa1)
        sb = (sb0, sb1)

        def start(i, slot):
            off = base + i * c
            pltpu.sync_copy(src_hbm.at[pl.ds(off, c)], idxs.at[slot])
            pltpu.sync_copy(dst_hbm.at[pl.ds(off, c)], idxd.at[slot])
            pltpu.async_copy(a_hbm.at[idxs.at[slot]], bufa.at[slot], sa[slot])
            pltpu.async_copy(b_hbm.at[idxd.at[slot]], bufb.at[slot], sb[slot])

        def drain(i, slot):
            off = base + i * c
            pltpu.make_async_copy(a_hbm.at[idxs.at[slot]], bufa.at[slot],
                                  sa[slot]).wait()
            pltpu.make_async_copy(b_hbm.at[idxd.at[slot]], bufb.at[slot],
                                  sb[slot]).wait()
            pltpu.sync_copy(bufa.at[slot], ga_hbm.at[pl.ds(off, c)])
            pltpu.sync_copy(bufb.at[slot], gb_hbm.at[pl.ds(off, c)])

        start(0, 0)
        start(1, 1)

        def body(j, carry):
            drain(2 * j, 0)
            start(2 * j + 2, 0)
            drain(2 * j + 1, 1)

            @pl.when(j < (nch - 3) // 2)
            def _():
                start(2 * j + 3, 1)

            return carry

        lax.fori_loop(0, (nch - 1) // 2, body, 0)
        drain(nch - 1, 0)

    return k(a, b, src, dst)


def _scatter_add(msg, dst, n):
    return jnp.zeros((n, msg.shape[1]), dtype=msg.dtype).at[dst].add(msg)


def kernel(x, edge_index, edge_attr, W_rbf, W_edge, W_msg, W_atom1, W_atom2, W_res):
    n, d = x.shape
    e = edge_index.shape[1]
    src = edge_index[0].astype(jnp.int32)
    dst = edge_index[1].astype(jnp.int32)
    w1 = W_edge[:d]
    w2 = W_edge[d:2 * d]
    w3 = W_edge[2 * d:]
    a, b = _node_proj(x, w1, w2)
    ga, gb = _sc_gather(a, b, src, dst)
    msg = _edge_msg(ga, gb, edge_attr, W_rbf, w3, W_msg)
    agg = _scatter_add(msg, dst, n)
    scale = 1.0 / math.sqrt(e / n)
    return _atom_update(agg, x, W_atom1, W_atom2, W_res, scale)
